# baseline (speedup 1.0000x reference)
import functools

import jax
import jax.numpy as jnp
from jax import lax
from jax.experimental import pallas as pl
from jax.experimental.pallas import tpu as pltpu
from jax.experimental.pallas import tpu_sc as plsc

_K = 50
_ROWS = 128
_N = 32768
_NW = 32
_RPW = _ROWS // _NW
_G = 32
_NMAX = _N // _G
_INT_MIN = -(2**31)


def _skey(v):
    b = lax.bitcast_convert_type(v, jnp.int32)
    return jnp.where(b >= 0, b, b ^ jnp.int32(0x7FFFFFFF))


def _count_ge_blocks(ref, nblk, c):
    cvec = jnp.full((16,), c, dtype=jnp.int32)

    def body(g, cnt):
        for j in range(8):
            v = ref[pl.ds(g * 128 + j * 16, 16)]
            cnt = cnt + jnp.where(v >= cvec, jnp.int32(1), jnp.int32(0))
        return cnt

    cnt_v = lax.fori_loop(0, nblk, body, jnp.zeros((16,), jnp.int32))
    return jnp.sum(cnt_v)


def _search_blocks(ref, nblk, k):
    def it(i, p):
        c = p + (jnp.int32(1) << (jnp.int32(31) - i))
        cnt = _count_ge_blocks(ref, nblk, c)
        return jnp.where(cnt >= k, c, p)

    return lax.fori_loop(0, 32, it, jnp.int32(_INT_MIN))


def _key_to_f32(kvv):
    bv = jnp.where(kvv >= 0, kvv, kvv ^ jnp.int32(0x7FFFFFFF))
    return lax.bitcast_convert_type(bv, jnp.float32)


def _sc_row(row_v, maxes_v, cand_v):
    def groups(g, _):
        base = g * (_G * 16)
        acc = row_v[pl.ds(base, 16)]
        for j in range(1, _G):
            acc = jnp.maximum(acc, row_v[pl.ds(base + j * 16, 16)])
        maxes_v[pl.ds(g * 16, 16)] = _skey(acc)
        return 0

    lax.fori_loop(0, _NMAX // 16, groups, 0)

    t0 = _search_blocks(maxes_v, _NMAX // 128, jnp.int32(_K))
    t0f = _key_to_f32(jnp.full((16,), t0, dtype=jnp.int32))

    def collect(g, off):
        vbs, ms, pcs = [], [], []
        for j in range(8):
            v = row_v[pl.ds(g * 128 + j * 16, 16)]
            m = v >= t0f
            vbs.append(lax.bitcast_convert_type(v, jnp.int32))
            ms.append(m)
            pcs.append(plsc.all_reduce_population_count(m)[0])
        offs = [off]
        for j in range(8):
            offs.append(offs[j] + pcs[j])
        for j in range(8):
            plsc.store_compressed(cand_v.at[pl.ds(offs[j], 16)], vbs[j], mask=ms[j])
        return offs[8]

    off = lax.fori_loop(0, _N // 128, collect, jnp.int32(0))

    sent = jnp.full((16,), jnp.int32(-1))
    for j in range(8):
        cand_v[pl.ds(off + j * 16, 16)] = sent

    nblk = (off + jnp.int32(127)) // jnp.int32(128)

    def conv(g, _):
        for j in range(8):
            s = pl.ds(g * 128 + j * 16, 16)
            b = cand_v[s]
            cand_v[s] = jnp.where(b >= 0, b, b ^ jnp.int32(0x7FFFFFFF))
        return 0

    lax.fori_loop(0, nblk, conv, 0)
    return _search_blocks(cand_v, nblk, jnp.int32(_K))


def _make_sc_thresholds(rpw):
    def _sc_body(x_hbm, out_hbm, row_a, row_b, maxes_v, cand_v, thr_v, sem):
        wid = lax.axis_index("s") * 2 + lax.axis_index("c")
        thr_v[...] = jnp.zeros((16,), jnp.float32)
        lane = jax.lax.broadcasted_iota(jnp.int32, (16,), 0)

        bufs = [row_a, row_b]
        pltpu.sync_copy(x_hbm.at[wid * rpw], bufs[0])
        for r in range(rpw):
            cp = None
            if r + 1 < rpw:
                cp = pltpu.async_copy(
                    x_hbm.at[wid * rpw + r + 1], bufs[(r + 1) % 2], sem
                )
            kv = _sc_row(bufs[r % 2], maxes_v, cand_v)
            tvv = _key_to_f32(jnp.full((16,), kv, dtype=jnp.int32))
            thr_v[...] = jnp.where(lane == r, tvv, thr_v[...])
            if cp is not None:
                cp.wait()

        pltpu.sync_copy(thr_v, out_hbm.at[wid])

    return functools.partial(
        pl.kernel,
        out_type=jax.ShapeDtypeStruct((_NW, 16), jnp.float32),
        mesh=plsc.VectorSubcoreMesh(core_axis_name="c", subcore_axis_name="s"),
        compiler_params=pltpu.CompilerParams(needs_layout_passes=False),
        scratch_types=[
            pltpu.VMEM((_N,), jnp.float32),
            pltpu.VMEM((_N,), jnp.float32),
            pltpu.VMEM((_NMAX,), jnp.int32),
            pltpu.VMEM((_N + 128,), jnp.int32),
            pltpu.VMEM((16,), jnp.float32),
            pltpu.SemaphoreType.DMA,
        ],
    )(_sc_body)


_SC_ROWS = 64
_sc_thresholds = _make_sc_thresholds(_SC_ROWS // _NW)


def _tc_count_cmp(v_s, c_s, strict):
    n = v_s.shape[1]
    w = n // 32
    one = jnp.bfloat16(1)
    zero = jnp.bfloat16(0)
    t = None
    for j in range(32):
        sl = v_s[:, j * w : (j + 1) * w]
        m = (sl > c_s) if strict else (sl >= c_s)
        part = jnp.where(m, one, zero)
        t = part if t is None else t + part
    while w > 128:
        half = w // 2
        t = t[:, :half] + t[:, half:]
        w = half
    f = t.astype(jnp.float32)
    return jnp.sum(f, axis=1, keepdims=True)


def _to_s16(c):
    return (c ^ jnp.int32(0x8000)).astype(jnp.int16)


def _tc_search16(v_s, target):
    rows = v_s.shape[0]
    p = jnp.zeros((rows, 1), dtype=jnp.int32)
    for i in range(15, -1, -1):
        c = p | jnp.int32(1 << i)
        cnt = _tc_count_cmp(v_s, _to_s16(c), strict=False)
        p = jnp.where(cnt >= target, c, p)
    return p


def _tc_kwta_kernel(x_ref, o_ref):
    x = x_ref[...]
    u = jax.lax.bitcast_convert_type(x, jnp.uint32)
    neg = (u >> 31) == 1
    ukey = jnp.where(neg, ~u, u | jnp.uint32(0x80000000))
    ikey = jax.lax.bitcast_convert_type(ukey, jnp.int32)

    hi_s = ((ikey >> 16) ^ jnp.int32(0x8000)).astype(jnp.int16)
    lo_s = (ikey ^ jnp.int32(0x8000)).astype(jnp.int16)
    k = jnp.float32(_K)

    p_hi = _tc_search16(hi_s, k)
    p_hi_s = _to_s16(p_hi)
    cnt_gt = _tc_count_cmp(hi_s, p_hi_s, strict=True)
    z_s = jnp.where(hi_s == p_hi_s, lo_s, jnp.int16(-32768))
    p_lo = _tc_search16(z_s, k - cnt_gt)

    kv = jax.lax.bitcast_convert_type((p_hi << 16) | p_lo, jnp.uint32)
    topbit = (kv >> 31) == 1
    u_orig = jnp.where(topbit, kv & jnp.uint32(0x7FFFFFFF), ~kv)
    tv = jax.lax.bitcast_convert_type(u_orig, jnp.float32)
    o_ref[...] = (x >= tv).astype(jnp.float32)


def _mask_kernel(x_ref, t_ref, o_ref):
    o_ref[...] = (x_ref[...] >= t_ref[...]).astype(jnp.float32)


@jax.jit
def kernel(x):
    m, n = x.shape
    ns = _SC_ROWS
    rpw = ns // _NW
    xa = x[:ns]
    xb = x[ns:]

    thr = _sc_thresholds(xa)
    r = 16
    maskb = pl.pallas_call(
        _tc_kwta_kernel,
        out_shape=jax.ShapeDtypeStruct((m - ns, n), jnp.float32),
        grid=((m - ns) // r,),
        in_specs=[pl.BlockSpec((r, n), lambda i: (i, 0))],
        out_specs=pl.BlockSpec((r, n), lambda i: (i, 0)),
    )(xb)

    thr_col = thr[:, :rpw].reshape(ns, 1)
    maska = pl.pallas_call(
        _mask_kernel,
        out_shape=jax.ShapeDtypeStruct((ns, n), jnp.float32),
        grid=(ns // r,),
        in_specs=[
            pl.BlockSpec((r, n), lambda i: (i, 0)),
            pl.BlockSpec((r, 1), lambda i: (i, 0)),
        ],
        out_specs=pl.BlockSpec((r, n), lambda i: (i, 0)),
    )(xa, thr_col)
    return jnp.concatenate([maska, maskb], axis=0)


# speedup vs baseline: 1.1099x; 1.1099x over previous
import functools

import jax
import jax.numpy as jnp
from jax import lax
from jax.experimental import pallas as pl
from jax.experimental.pallas import tpu as pltpu
from jax.experimental.pallas import tpu_sc as plsc

_K = 50
_ROWS = 128
_N = 32768
_NW = 32
_RPW = _ROWS // _NW
_G = 32
_NMAX = _N // _G
_INT_MIN = -(2**31)


def _skey(v):
    b = lax.bitcast_convert_type(v, jnp.int32)
    return jnp.where(b >= 0, b, b ^ jnp.int32(0x7FFFFFFF))


def _count_ge_blocks(ref, nblk, c):
    cvec = jnp.full((16,), c, dtype=jnp.int32)

    def body(g, cnt):
        for j in range(8):
            v = ref[pl.ds(g * 128 + j * 16, 16)]
            cnt = cnt + jnp.where(v >= cvec, jnp.int32(1), jnp.int32(0))
        return cnt

    cnt_v = lax.fori_loop(0, nblk, body, jnp.zeros((16,), jnp.int32))
    return jnp.sum(cnt_v)


def _search_blocks(ref, nblk, k):
    def it(i, p):
        c = p + (jnp.int32(1) << (jnp.int32(31) - i))
        cnt = _count_ge_blocks(ref, nblk, c)
        return jnp.where(cnt >= k, c, p)

    return lax.fori_loop(0, 32, it, jnp.int32(_INT_MIN))


def _key_to_f32(kvv):
    bv = jnp.where(kvv >= 0, kvv, kvv ^ jnp.int32(0x7FFFFFFF))
    return lax.bitcast_convert_type(bv, jnp.float32)


def _fused_sweep(row_v, maxes_v, cand_v, t0f):
    def body(g, off):
        acc = None
        for sb in range(4):
            vbs, ms, pcs = [], [], []
            for j in range(8):
                v = row_v[pl.ds(g * 512 + (sb * 8 + j) * 16, 16)]
                acc = v if acc is None else jnp.maximum(acc, v)
                m = v >= t0f
                vbs.append(lax.bitcast_convert_type(v, jnp.int32))
                ms.append(m)
                pcs.append(plsc.all_reduce_population_count(m)[0])
            offs = [off]
            for j in range(8):
                offs.append(offs[j] + pcs[j])
            for j in range(8):
                plsc.store_compressed(
                    cand_v.at[pl.ds(offs[j], 16)], vbs[j], mask=ms[j]
                )
            off = offs[8]
        maxes_v[pl.ds(g * 16, 16)] = _skey(acc)
        return off

    return lax.fori_loop(0, _N // 512, body, jnp.int32(0))


def _collect_blocks(row_v, cand_v, t0f):
    def collect(g, off):
        vbs, ms, pcs = [], [], []
        for j in range(8):
            v = row_v[pl.ds(g * 128 + j * 16, 16)]
            m = v >= t0f
            vbs.append(lax.bitcast_convert_type(v, jnp.int32))
            ms.append(m)
            pcs.append(plsc.all_reduce_population_count(m)[0])
        offs = [off]
        for j in range(8):
            offs.append(offs[j] + pcs[j])
        for j in range(8):
            plsc.store_compressed(cand_v.at[pl.ds(offs[j], 16)], vbs[j], mask=ms[j])
        return offs[8]

    return lax.fori_loop(0, _N // 128, collect, jnp.int32(0))


def _finish(cand_v, off):
    sent = jnp.full((16,), jnp.int32(-1))
    for j in range(8):
        cand_v[pl.ds(off + j * 16, 16)] = sent

    nblk = (off + jnp.int32(127)) // jnp.int32(128)

    def conv(g, _):
        for j in range(8):
            s = pl.ds(g * 128 + j * 16, 16)
            b = cand_v[s]
            cand_v[s] = jnp.where(b >= 0, b, b ^ jnp.int32(0x7FFFFFFF))
        return 0

    lax.fori_loop(0, nblk, conv, 0)
    return _search_blocks(cand_v, nblk, jnp.int32(_K))


def _sc_body(x_hbm, out_hbm, row_a, row_b, maxes_v, cand_v, thr_v, sem):
    wid = lax.axis_index("s") * 2 + lax.axis_index("c")
    thr_v[...] = jnp.zeros((16,), jnp.float32)
    lane = jax.lax.broadcasted_iota(jnp.int32, (16,), 0)

    bufs = [row_a, row_b]
    pltpu.sync_copy(x_hbm.at[wid * _RPW], bufs[0])
    t_try = jnp.int32(0x7F7FFFFF)
    for r in range(_RPW):
        cp = None
        if r + 1 < _RPW:
            cp = pltpu.async_copy(
                x_hbm.at[wid * _RPW + r + 1], bufs[(r + 1) % 2], sem
            )
        buf = bufs[r % 2]
        t_tryf = _key_to_f32(jnp.full((16,), t_try, dtype=jnp.int32))
        off1 = _fused_sweep(buf, maxes_v, cand_v, t_tryf)

        def _redo(_):
            t0 = _search_blocks(maxes_v, _NMAX // 128, jnp.int32(_K))
            t0f = _key_to_f32(jnp.full((16,), t0, dtype=jnp.int32))
            return _collect_blocks(buf, cand_v, t0f)

        off = lax.cond(off1 >= _K, lambda _: off1, _redo, 0)
        kv = _finish(cand_v, off)
        t_try = kv - jnp.int32(1 << 20)
        tvv = _key_to_f32(jnp.full((16,), kv, dtype=jnp.int32))
        thr_v[...] = jnp.where(lane == r, tvv, thr_v[...])
        if cp is not None:
            cp.wait()

    pltpu.sync_copy(thr_v, out_hbm.at[wid])


_sc_thresholds = functools.partial(
    pl.kernel,
    out_type=jax.ShapeDtypeStruct((_NW, 16), jnp.float32),
    mesh=plsc.VectorSubcoreMesh(core_axis_name="c", subcore_axis_name="s"),
    compiler_params=pltpu.CompilerParams(needs_layout_passes=False),
    scratch_types=[
        pltpu.VMEM((_N,), jnp.float32),
        pltpu.VMEM((_N,), jnp.float32),
        pltpu.VMEM((_NMAX,), jnp.int32),
        pltpu.VMEM((_N + 128,), jnp.int32),
        pltpu.VMEM((16,), jnp.float32),
        pltpu.SemaphoreType.DMA,
    ],
)(_sc_body)


def _mask_kernel(x_ref, t_ref, o_ref):
    o_ref[...] = (x_ref[...] >= t_ref[...]).astype(jnp.float32)


@jax.jit
def kernel(x):
    m, n = x.shape
    thr = _sc_thresholds(x)
    thr_col = thr[:, :_RPW].reshape(m, 1)
    r = 16
    return pl.pallas_call(
        _mask_kernel,
        out_shape=jax.ShapeDtypeStruct((m, n), jnp.float32),
        grid=(m // r,),
        in_specs=[
            pl.BlockSpec((r, n), lambda i: (i, 0)),
            pl.BlockSpec((r, 1), lambda i: (i, 0)),
        ],
        out_specs=pl.BlockSpec((r, n), lambda i: (i, 0)),
    )(x, thr_col)
